# baseline (speedup 1.0000x reference)
import functools

import numpy as np
import jax
import jax.numpy as jnp
from jax import lax
from jax.experimental import pallas as pl
from jax.experimental.pallas import tpu as pltpu
from jax.experimental.pallas import tpu_sc as plsc

D_MODEL = 128
MEMORY_LEN = 200
BATCH = 4096
NUM_COMBO = 25
SCALE = float(np.sqrt(D_MODEL))
EPS = 1e-5
ROWS = MEMORY_LEN * BATCH
WINDOW = 128
GRID = ROWS // WINDOW


def _prep_body(pos_ref, pt_ref, lt_ref, g_ref, b_ref, xp_ref, xl_ref,
               t_ref, combo_ref):
    gamma = g_ref[...]
    beta = b_ref[...]
    pos2 = 2.0 * pos_ref[...]
    for c in range(NUM_COMBO):
        p, lab = c // 5, c % 5
        row = SCALE * (pt_ref[p:p + 1, :] + lt_ref[lab:lab + 1, :])
        x = pos2 + row
        mean = jnp.mean(x, axis=-1, keepdims=True)
        var = jnp.mean((x - mean) ** 2, axis=-1, keepdims=True)
        t_ref[c] = (x - mean) * lax.rsqrt(var + EPS) * gamma + beta
    iota_l = lax.broadcasted_iota(jnp.int32, (BATCH, MEMORY_LEN), 1)
    combo_ref[...] = (5 * xp_ref[...] + xl_ref[...]) * MEMORY_LEN + iota_l


def _prep(pos_table, pitch_table, label_table, ln_gamma, ln_beta, xp_t, xl_t):
    return pl.pallas_call(
        _prep_body,
        out_shape=[
            jax.ShapeDtypeStruct((NUM_COMBO, MEMORY_LEN, D_MODEL), jnp.float32),
            jax.ShapeDtypeStruct((BATCH, MEMORY_LEN), jnp.int32),
        ],
    )(pos_table, pitch_table, label_table, ln_gamma, ln_beta, xp_t, xl_t)


def _gather(table, idx):
    mesh = plsc.VectorSubcoreMesh(core_axis_name="core",
                                  subcore_axis_name="subcore")

    @functools.partial(
        pl.kernel,
        out_type=jax.ShapeDtypeStruct((ROWS, D_MODEL), jnp.float32),
        mesh=mesh,
        scratch_types=[
            pltpu.VMEM_SHARED((NUM_COMBO * MEMORY_LEN, D_MODEL), jnp.float32),
        ],
    )
    def k(t_hbm, i_hbm, o_hbm, t_spmem):
        sid = lax.axis_index("subcore")

        @pl.when(sid < 15)
        def _():
            pltpu.sync_copy(t_hbm.at[pl.ds(sid * 312, 312)],
                            t_spmem.at[pl.ds(sid * 312, 312)])

        @pl.when(sid == 15)
        def _():
            pltpu.sync_copy(t_hbm.at[pl.ds(4680, 320)],
                            t_spmem.at[pl.ds(4680, 320)])

        plsc.subcore_barrier()

        def body(i_vmem, o_vmem):
            pltpu.sync_copy(t_spmem.at[i_vmem.at[0]],
                            o_vmem.at[pl.ds(0, WINDOW)])
            pltpu.sync_copy(t_spmem.at[i_vmem.at[1]],
                            o_vmem.at[pl.ds(WINDOW, WINDOW)])

        pltpu.emit_pipeline(
            body,
            grid=(GRID // 2,),
            in_specs=[pl.BlockSpec((2, WINDOW), index_map=lambda i: (i, 0))],
            out_specs=[pl.BlockSpec((2 * WINDOW, D_MODEL),
                                    index_map=lambda i: (i, 0))],
            core_axis_name=("core", "subcore"),
            dimension_semantics=(pltpu.PARALLEL,),
        )(i_hbm, o_hbm)

    return k(table, idx)


def kernel(x_pitch, x_label, pos_table, pitch_table, label_table,
           ln_gamma, ln_beta):
    t, combo_bl = _prep(pos_table, pitch_table, label_table,
                        ln_gamma.reshape(1, D_MODEL),
                        ln_beta.reshape(1, D_MODEL), x_pitch, x_label)
    out_flat = _gather(t.reshape(NUM_COMBO * MEMORY_LEN, D_MODEL),
                       combo_bl.T.reshape(GRID, WINDOW))
    return out_flat.reshape(MEMORY_LEN, BATCH, D_MODEL)


# speedup vs baseline: 1.0570x; 1.0570x over previous
import functools

import numpy as np
import jax
import jax.numpy as jnp
from jax import lax
from jax.experimental import pallas as pl
from jax.experimental.pallas import tpu as pltpu
from jax.experimental.pallas import tpu_sc as plsc

D_MODEL = 128
MEMORY_LEN = 200
BATCH = 4096
NUM_COMBO = 25
SCALE = float(np.sqrt(D_MODEL))
EPS = 1e-5
ROWS = MEMORY_LEN * BATCH
WINDOW = 128
GRID = ROWS // WINDOW


def _prep_body(pos_ref, pt_ref, lt_ref, g_ref, b_ref, xp_ref, xl_ref,
               t_ref, combo_ref):
    gamma = g_ref[...]
    beta = b_ref[...]
    pos2 = 2.0 * pos_ref[...]
    for c in range(NUM_COMBO):
        p, lab = c // 5, c % 5
        row = SCALE * (pt_ref[p:p + 1, :] + lt_ref[lab:lab + 1, :])
        x = pos2 + row
        mean = jnp.mean(x, axis=-1, keepdims=True)
        var = jnp.mean((x - mean) ** 2, axis=-1, keepdims=True)
        t_ref[c] = (x - mean) * lax.rsqrt(var + EPS) * gamma + beta
    iota_l = lax.broadcasted_iota(jnp.int32, (MEMORY_LEN, BATCH), 0)
    combo_ref[...] = (5 * xp_ref[...] + xl_ref[...]) * MEMORY_LEN + iota_l


def _prep(pos_table, pitch_table, label_table, ln_gamma, ln_beta, xp_t, xl_t):
    return pl.pallas_call(
        _prep_body,
        out_shape=[
            jax.ShapeDtypeStruct((NUM_COMBO, MEMORY_LEN, D_MODEL), jnp.float32),
            jax.ShapeDtypeStruct((MEMORY_LEN, BATCH), jnp.int32),
        ],
    )(pos_table, pitch_table, label_table, ln_gamma, ln_beta, xp_t, xl_t)


def _gather(table, idx):
    mesh = plsc.VectorSubcoreMesh(core_axis_name="core",
                                  subcore_axis_name="subcore")

    @functools.partial(
        pl.kernel,
        out_type=jax.ShapeDtypeStruct((ROWS, D_MODEL), jnp.float32),
        mesh=mesh,
        scratch_types=[
            pltpu.VMEM_SHARED((NUM_COMBO * MEMORY_LEN, D_MODEL), jnp.float32),
        ],
    )
    def k(t_hbm, i_hbm, o_hbm, t_spmem):
        sid = lax.axis_index("subcore")

        @pl.when(sid < 15)
        def _():
            pltpu.sync_copy(t_hbm.at[pl.ds(sid * 312, 312)],
                            t_spmem.at[pl.ds(sid * 312, 312)])

        @pl.when(sid == 15)
        def _():
            pltpu.sync_copy(t_hbm.at[pl.ds(4680, 320)],
                            t_spmem.at[pl.ds(4680, 320)])

        plsc.subcore_barrier()

        def body(i_vmem, o_vmem):
            pltpu.sync_copy(t_spmem.at[i_vmem.at[0]],
                            o_vmem.at[pl.ds(0, WINDOW)])
            pltpu.sync_copy(t_spmem.at[i_vmem.at[1]],
                            o_vmem.at[pl.ds(WINDOW, WINDOW)])

        pltpu.emit_pipeline(
            body,
            grid=(GRID // 2,),
            in_specs=[pl.BlockSpec((2, WINDOW), index_map=lambda i: (i, 0))],
            out_specs=[pl.BlockSpec((2 * WINDOW, D_MODEL),
                                    index_map=lambda i: (i, 0))],
            core_axis_name=("core", "subcore"),
            dimension_semantics=(pltpu.PARALLEL,),
        )(i_hbm, o_hbm)

    return k(table, idx)


def kernel(x_pitch, x_label, pos_table, pitch_table, label_table,
           ln_gamma, ln_beta):
    xp_t = x_pitch.T
    xl_t = x_label.T
    t, combo = _prep(pos_table, pitch_table, label_table,
                     ln_gamma.reshape(1, D_MODEL), ln_beta.reshape(1, D_MODEL),
                     xp_t, xl_t)
    out_flat = _gather(t.reshape(NUM_COMBO * MEMORY_LEN, D_MODEL),
                       combo.reshape(GRID, WINDOW))
    return out_flat.reshape(MEMORY_LEN, BATCH, D_MODEL)


# speedup vs baseline: 1.1159x; 1.0557x over previous
import functools

import numpy as np
import jax
import jax.numpy as jnp
from jax import lax
from jax.experimental import pallas as pl
from jax.experimental.pallas import tpu as pltpu
from jax.experimental.pallas import tpu_sc as plsc

D_MODEL = 128
MEMORY_LEN = 200
BATCH = 4096
NUM_COMBO = 25
SCALE = float(np.sqrt(D_MODEL))
EPS = 1e-5
ROWS = MEMORY_LEN * BATCH
WINDOW = 128
GRID = ROWS // WINDOW


def _prep_body(pos_ref, pt_ref, lt_ref, g_ref, b_ref, xp_ref, xl_ref,
               t_ref, combo_ref):
    gamma = g_ref[...]
    beta = b_ref[...]
    pos2 = 2.0 * pos_ref[...]
    for c in range(NUM_COMBO):
        p, lab = c // 5, c % 5
        row = SCALE * (pt_ref[p:p + 1, :] + lt_ref[lab:lab + 1, :])
        x = pos2 + row
        mean = jnp.mean(x, axis=-1, keepdims=True)
        var = jnp.mean((x - mean) ** 2, axis=-1, keepdims=True)
        t_ref[c] = (x - mean) * lax.rsqrt(var + EPS) * gamma + beta
    iota_l = lax.broadcasted_iota(jnp.int32, (MEMORY_LEN, BATCH), 0)
    combo_ref[...] = (5 * xp_ref[...] + xl_ref[...]) * MEMORY_LEN + iota_l


def _prep(pos_table, pitch_table, label_table, ln_gamma, ln_beta, xp_t, xl_t):
    return pl.pallas_call(
        _prep_body,
        out_shape=[
            jax.ShapeDtypeStruct((NUM_COMBO, MEMORY_LEN, D_MODEL), jnp.float32),
            jax.ShapeDtypeStruct((MEMORY_LEN, BATCH), jnp.int32),
        ],
    )(pos_table, pitch_table, label_table, ln_gamma, ln_beta, xp_t, xl_t)


def _gather(table, idx):
    mesh = plsc.VectorSubcoreMesh(core_axis_name="core",
                                  subcore_axis_name="subcore")

    @functools.partial(
        pl.kernel,
        out_type=jax.ShapeDtypeStruct((ROWS, D_MODEL), jnp.float32),
        mesh=mesh,
        scratch_types=[
            pltpu.VMEM_SHARED((NUM_COMBO * MEMORY_LEN, D_MODEL), jnp.float32),
            pltpu.SemaphoreType.DMA,
            pltpu.SemaphoreType.DMA,
        ],
    )
    def k(t_hbm, i_hbm, o_hbm, t_spmem, sem_a, sem_b):
        sid = lax.axis_index("subcore")

        @pl.when(sid < 15)
        def _():
            pltpu.sync_copy(t_hbm.at[pl.ds(sid * 312, 312)],
                            t_spmem.at[pl.ds(sid * 312, 312)])

        @pl.when(sid == 15)
        def _():
            pltpu.sync_copy(t_hbm.at[pl.ds(4680, 320)],
                            t_spmem.at[pl.ds(4680, 320)])

        plsc.subcore_barrier()

        def body(i_vmem, o_vmem):
            cp_a = pltpu.async_copy(t_spmem.at[i_vmem.at[0]],
                                    o_vmem.at[pl.ds(0, WINDOW)], sem_a)
            cp_b = pltpu.async_copy(t_spmem.at[i_vmem.at[1]],
                                    o_vmem.at[pl.ds(WINDOW, WINDOW)], sem_b)
            cp_a.wait()
            cp_b.wait()

        pltpu.emit_pipeline(
            body,
            grid=(GRID // 2,),
            in_specs=[pl.BlockSpec((2, WINDOW), index_map=lambda i: (i, 0))],
            out_specs=[pl.BlockSpec((2 * WINDOW, D_MODEL),
                                    index_map=lambda i: (i, 0))],
            core_axis_name=("core", "subcore"),
            dimension_semantics=(pltpu.PARALLEL,),
        )(i_hbm, o_hbm)

    return k(table, idx)


def kernel(x_pitch, x_label, pos_table, pitch_table, label_table,
           ln_gamma, ln_beta):
    xp_t = x_pitch.T
    xl_t = x_label.T
    t, combo = _prep(pos_table, pitch_table, label_table,
                     ln_gamma.reshape(1, D_MODEL), ln_beta.reshape(1, D_MODEL),
                     xp_t, xl_t)
    out_flat = _gather(t.reshape(NUM_COMBO * MEMORY_LEN, D_MODEL),
                       combo.reshape(GRID, WINDOW))
    return out_flat.reshape(MEMORY_LEN, BATCH, D_MODEL)
